# Initial kernel scaffold; baseline (speedup 1.0000x reference)
import jax
import jax.numpy as jnp
from jax.experimental import pallas as pl


def kernel(x, graph, norm1_w, norm1_b, W_qkv, W_proj, b_proj, norm2_w, norm2_b, W_fc1, b_fc1, W_fc2, b_fc2):
    raise NotImplementedError("write your pallas kernel here")


# speedup vs baseline: 2.1339x; 2.1339x over previous
import jax
import jax.numpy as jnp
from jax.experimental import pallas as pl
from jax.experimental.pallas import tpu as pltpu

DIM = 768
HEADS = 12
HD = 64
HIDDEN = 3072
NUM_PROP = 128
ALPHA = 0.1
N = 577
NT = N - 1
NP = 640
KEEP = NT - NUM_PROP
NOUT = 1 + KEEP
NOP = 456
EPS = 1e-5
SCALE = HD ** -0.5
NEG = -1e30


def _ln(x, w, b):
    mu = x.mean(-1, keepdims=True)
    var = ((x - mu) ** 2).mean(-1, keepdims=True)
    return (x - mu) / jnp.sqrt(var + EPS) * w + b


def _ln_qkv_kernel(x_ref, w_ref, b_ref, wqkv_ref, qkv_ref):
    h = _ln(x_ref[0], w_ref[0], b_ref[0])
    qkv_ref[0, 0] = jnp.dot(h, wqkv_ref[0],
                            preferred_element_type=jnp.float32)


def _attn_kernel(x_ref, q_ref, k_ref, v_ref, wp_ref, bp_ref, o_ref, st_ref):
    h = pl.program_id(1)
    q = q_ref[0, 0] * SCALE
    k = k_ref[0, 0]
    v = v_ref[0, 0]
    s = jax.lax.dot_general(q, k, (((1,), (1,)), ((), ())),
                            preferred_element_type=jnp.float32)
    col = jax.lax.broadcasted_iota(jnp.int32, (NP, NP), 1)
    row = jax.lax.broadcasted_iota(jnp.int32, (NP, NP), 0)
    s = jnp.where(col < N, s, NEG)
    m = s.max(-1, keepdims=True)
    e = jnp.exp(s - m)
    p = e / e.sum(-1, keepdims=True)
    st_ref[0, 0, 0] = jnp.where(row == col, p, 0.0).sum(0)
    st_ref[0, 0, 1] = jnp.where(row < N, p, 0.0).sum(0)
    pv = jnp.dot(p, v, preferred_element_type=jnp.float32)
    contrib = jnp.dot(pv, wp_ref[...], preferred_element_type=jnp.float32)

    @pl.when(h == 0)
    def _():
        o_ref[0] = x_ref[0] + bp_ref[0] + contrib

    @pl.when(h > 0)
    def _():
        o_ref[0] = o_ref[0] + contrib

    @pl.when(h == HEADS - 1)
    def _():
        rmask = jax.lax.broadcasted_iota(jnp.int32, (NP, 1), 0) < N
        o_ref[0] = jnp.where(rmask, o_ref[0], 0.0)


def _prop_kernel(st_ref, x_ref, g_ref, xn_ref, wk_ref):
    d = st_ref[0, :, 0, :]
    c = st_ref[0, :, 1, :]
    tr1 = d.mean(0, keepdims=True)
    tr2 = c.mean(0, keepdims=True)
    ii = jax.lax.broadcasted_iota(jnp.int32, (1, NP), 1)
    v = jnp.where((ii >= 1) & (ii < N), tr1 * tr2, NEG)
    vb = jnp.broadcast_to(v, (NP, NP))
    vt = vb.T
    row = jax.lax.broadcasted_iota(jnp.int32, (NP, NP), 0)
    col = jax.lax.broadcasted_iota(jnp.int32, (NP, NP), 1)
    beats = (vt > vb) | ((vt == vb) & (row < col))
    rank = beats.astype(jnp.float32).sum(0)
    perm = (row == rank[None, :].astype(jnp.int32)).astype(jnp.float32)
    pk = perm[:KEEP]
    pe = perm[KEEP:NT]
    xr = x_ref[0]
    g = g_ref[0]
    xk = jnp.dot(pk, xr, preferred_element_type=jnp.float32)
    xe = jnp.dot(pe, xr, preferred_element_type=jnp.float32)
    w = jnp.dot(pk, g, preferred_element_type=jnp.float32)
    wp = jax.lax.dot_general(w, pe, (((1,), (1,)), ((), ())),
                             preferred_element_type=jnp.float32)
    wk_ref[0] = jax.lax.dot_general(w, pk, (((1,), (1,)), ((), ())),
                                    preferred_element_type=jnp.float32)
    xk = xk + ALPHA * jnp.dot(wp, xe, preferred_element_type=jnp.float32)
    xn_ref[0] = jnp.concatenate(
        [xr[0:1], xk, jnp.zeros((NOP - NOUT, DIM), jnp.float32)], axis=0)


def _mlp_kernel(x_ref, w2_ref, b2_ref, wf1_ref, bf1_ref, wf2_ref, bf2_ref, o_ref):
    xx = x_ref[0]
    h = _ln(xx, w2_ref[0], b2_ref[0])
    a = jnp.dot(h, wf1_ref[...], preferred_element_type=jnp.float32) + bf1_ref[0]
    ge = 0.5 * a * (1.0 + jax.lax.erf(a * (2.0 ** -0.5)))
    o_ref[0] = jnp.dot(ge, wf2_ref[...], preferred_element_type=jnp.float32) \
        + bf2_ref[0] + xx


def kernel(x, graph, norm1_w, norm1_b, W_qkv, W_proj, b_proj,
           norm2_w, norm2_b, W_fc1, b_fc1, W_fc2, b_fc2):
    B = x.shape[0]
    f32 = jnp.float32
    xp = jnp.pad(x, ((0, 0), (0, NP - N), (0, 0)))
    gp = jnp.pad(graph, ((0, 0), (1, NP - N), (1, NP - N)))
    n1w = norm1_w.reshape(1, DIM)
    n1b = norm1_b.reshape(1, DIM)
    n2w = norm2_w.reshape(1, DIM)
    n2b = norm2_b.reshape(1, DIM)
    bp = b_proj.reshape(1, DIM)
    bf1 = b_fc1.reshape(1, HIDDEN)
    bf2 = b_fc2.reshape(1, DIM)

    nchunk = 3 * HEADS
    wq3 = W_qkv.reshape(DIM, nchunk, HD).transpose(1, 0, 2)
    qkv = pl.pallas_call(
        _ln_qkv_kernel,
        grid=(B, nchunk),
        in_specs=[
            pl.BlockSpec((1, NP, DIM), lambda b, c: (b, 0, 0)),
            pl.BlockSpec((1, DIM), lambda b, c: (0, 0)),
            pl.BlockSpec((1, DIM), lambda b, c: (0, 0)),
            pl.BlockSpec((1, DIM, HD), lambda b, c: (c, 0, 0)),
        ],
        out_specs=pl.BlockSpec((1, 1, NP, HD), lambda b, c: (b, c, 0, 0)),
        out_shape=jax.ShapeDtypeStruct((B, nchunk, NP, HD), f32),
        compiler_params=pltpu.CompilerParams(
            dimension_semantics=("parallel", "arbitrary")),
    )(xp, n1w, n1b, wq3)

    xres, stats = pl.pallas_call(
        _attn_kernel,
        grid=(B, HEADS),
        in_specs=[
            pl.BlockSpec((1, NP, DIM), lambda b, h: (b, 0, 0)),
            pl.BlockSpec((1, 1, NP, HD), lambda b, h: (b, h, 0, 0)),
            pl.BlockSpec((1, 1, NP, HD), lambda b, h: (b, HEADS + h, 0, 0)),
            pl.BlockSpec((1, 1, NP, HD), lambda b, h: (b, 2 * HEADS + h, 0, 0)),
            pl.BlockSpec((HD, DIM), lambda b, h: (h, 0)),
            pl.BlockSpec((1, DIM), lambda b, h: (0, 0)),
        ],
        out_specs=[
            pl.BlockSpec((1, NP, DIM), lambda b, h: (b, 0, 0)),
            pl.BlockSpec((1, 1, 16, NP), lambda b, h: (b, h, 0, 0)),
        ],
        out_shape=[
            jax.ShapeDtypeStruct((B, NP, DIM), f32),
            jax.ShapeDtypeStruct((B, HEADS, 16, NP), f32),
        ],
        compiler_params=pltpu.CompilerParams(
            dimension_semantics=("parallel", "arbitrary")),
    )(xp, qkv, qkv, qkv, W_proj, bp)

    xnew, weight_kept = pl.pallas_call(
        _prop_kernel,
        grid=(B,),
        in_specs=[
            pl.BlockSpec((1, HEADS, 16, NP), lambda b: (b, 0, 0, 0)),
            pl.BlockSpec((1, NP, DIM), lambda b: (b, 0, 0)),
            pl.BlockSpec((1, NP, NP), lambda b: (b, 0, 0)),
        ],
        out_specs=[
            pl.BlockSpec((1, NOP, DIM), lambda b: (b, 0, 0)),
            pl.BlockSpec((1, KEEP, KEEP), lambda b: (b, 0, 0)),
        ],
        out_shape=[
            jax.ShapeDtypeStruct((B, NOP, DIM), f32),
            jax.ShapeDtypeStruct((B, KEEP, KEEP), f32),
        ],
        compiler_params=pltpu.CompilerParams(
            dimension_semantics=("parallel",)),
    )(stats, xres, gp)

    out = pl.pallas_call(
        _mlp_kernel,
        grid=(B,),
        in_specs=[
            pl.BlockSpec((1, NOP, DIM), lambda b: (b, 0, 0)),
            pl.BlockSpec((1, DIM), lambda b: (0, 0)),
            pl.BlockSpec((1, DIM), lambda b: (0, 0)),
            pl.BlockSpec((DIM, HIDDEN), lambda b: (0, 0)),
            pl.BlockSpec((1, HIDDEN), lambda b: (0, 0)),
            pl.BlockSpec((HIDDEN, DIM), lambda b: (0, 0)),
            pl.BlockSpec((1, DIM), lambda b: (0, 0)),
        ],
        out_specs=pl.BlockSpec((1, NOP, DIM), lambda b: (b, 0, 0)),
        out_shape=jax.ShapeDtypeStruct((B, NOP, DIM), f32),
        compiler_params=pltpu.CompilerParams(
            dimension_semantics=("parallel",)),
    )(xnew, n2w, n2b, W_fc1, bf1, W_fc2, bf2)

    return out[:, :NOUT], weight_kept
